# Initial kernel scaffold; baseline (speedup 1.0000x reference)
import jax
import jax.numpy as jnp
from jax.experimental import pallas as pl


def kernel(center, context, negative_samples, W_center, W_context):
    raise NotImplementedError("write your pallas kernel here")


# speedup vs baseline: 4.3159x; 4.3159x over previous
import functools

import jax
import jax.numpy as jnp
from jax import lax
from jax.experimental import pallas as pl
from jax.experimental.pallas import tpu as pltpu
from jax.experimental.pallas import tpu_sc as plsc

B = 16384
D = 64
N_NEG = 20
NS_TOT = N_NEG + 1
NC, NSUB, L = 2, 16, 16
NW = NC * NSUB
BP = B // NW
MB = 16
NMACRO = BP // MB
NEG_CHUNK = 80
NEG_ROWS_PER_MACRO = MB * N_NEG // NEG_CHUNK


def _sc_scores(center_rs, context_rs, neg_rs, W_center, W_context):
    mesh = plsc.VectorSubcoreMesh(core_axis_name="c", subcore_axis_name="s")

    @functools.partial(
        pl.kernel,
        out_type=jax.ShapeDtypeStruct((B * NS_TOT,), jnp.float32),
        mesh=mesh,
        scratch_types=[
            pltpu.VMEM((NMACRO, MB), jnp.int32),
            pltpu.VMEM((NMACRO, MB), jnp.int32),
            pltpu.VMEM((NMACRO * NEG_ROWS_PER_MACRO, NEG_CHUNK), jnp.int32),
            pltpu.VMEM((MB, D), jnp.float32),
            pltpu.VMEM((MB, D), jnp.float32),
            pltpu.VMEM((MB * N_NEG, D), jnp.float32),
            pltpu.VMEM((MB * NS_TOT,), jnp.float32),
            pltpu.SemaphoreType.DMA,
        ],
        compiler_params=pltpu.CompilerParams(
            needs_layout_passes=False, use_tc_tiling_on_sc=False),
    )
    def k(center_ref, context_ref, neg_ref, wc_ref, wx_ref, out_ref,
          cidx, xidx, nidx, cbuf, xbuf, nbuf, sbuf, sem):
        w = lax.axis_index("s") * NC + lax.axis_index("c")

        pltpu.sync_copy(center_ref.at[pl.ds(w * NMACRO, NMACRO)], cidx)
        pltpu.sync_copy(context_ref.at[pl.ds(w * NMACRO, NMACRO)], xidx)
        nrows = NMACRO * NEG_ROWS_PER_MACRO
        pltpu.sync_copy(neg_ref.at[pl.ds(w * nrows, nrows)], nidx)

        iota = lax.iota(jnp.int32, L)
        rows_b = iota
        rows_n = [iota * N_NEG + n for n in range(N_NEG)]
        sidx = [iota * NS_TOT + n for n in range(NS_TOT)]

        def macro(m, carry):
            copies = [
                pltpu.async_copy(wc_ref.at[cidx.at[m]], cbuf, sem),
                pltpu.async_copy(wx_ref.at[xidx.at[m]], xbuf, sem),
            ]
            for j in range(NEG_ROWS_PER_MACRO):
                copies.append(pltpu.async_copy(
                    wx_ref.at[nidx.at[m * NEG_ROWS_PER_MACRO + j]],
                    nbuf.at[pl.ds(j * NEG_CHUNK, NEG_CHUNK)], sem))
            for c in copies:
                c.wait()

            acc = [jnp.zeros((L,), jnp.float32) for _ in range(NS_TOT)]
            for g in range(D // L):
                cols = [jnp.full((L,), g * L + kk, jnp.int32)
                        for kk in range(L)]
                ct = [plsc.load_gather(cbuf, [rows_b, cols[kk]])
                      for kk in range(L)]
                for kk in range(L):
                    xv = plsc.load_gather(xbuf, [rows_b, cols[kk]])
                    acc[0] = acc[0] + ct[kk] * xv
                for n in range(N_NEG):
                    for kk in range(L):
                        nv = plsc.load_gather(nbuf, [rows_n[n], cols[kk]])
                        acc[n + 1] = acc[n + 1] - ct[kk] * nv
            for n in range(NS_TOT):
                plsc.store_scatter(sbuf, [sidx[n]], acc[n])
            pltpu.sync_copy(
                sbuf,
                out_ref.at[pl.ds(w * BP * NS_TOT + m * MB * NS_TOT,
                                 MB * NS_TOT)])
            return carry

        lax.fori_loop(0, NMACRO, macro, 0)

    return k(center_rs, context_rs, neg_rs, W_center, W_context)


def _loss_tc(scores_flat):
    s2 = scores_flat.reshape(B * NS_TOT // 128, 128)

    def body(s_ref, o_ref):
        x = s_ref[...]
        l = jnp.log(jax.nn.sigmoid(x) + 1e-10)
        o_ref[...] = (-jnp.sum(l) / B).reshape(1, 1)

    out = pl.pallas_call(
        body,
        out_shape=jax.ShapeDtypeStruct((1, 1), jnp.float32),
    )(s2)
    return out[0, 0]


def kernel(center, context, negative_samples, W_center, W_context):
    center_rs = center.reshape(B // MB, MB)
    context_rs = context.reshape(B // MB, MB)
    neg_rs = negative_samples.reshape(B * N_NEG // NEG_CHUNK, NEG_CHUNK)
    scores = _sc_scores(center_rs, context_rs, neg_rs, W_center, W_context)
    return _loss_tc(scores)
